# baseline (device time: 14305 ns/iter reference)
import jax
import jax.numpy as jnp
from jax import lax
from jax.experimental import pallas as pl
from jax.experimental.pallas import tpu as pltpu

N_DEV = 8
EPS = 1e-5


def kernel(x, t_emb, W_scale, W_shift):
    b, s, c_sh = x.shape
    c_glob = c_sh * N_DEV

    def body(x_ref, t_ref, wsc_ref, wsh_ref, out_ref,
             stats_ref, send_sems, recv_sems):
        my = lax.axis_index("i")

        barrier = pltpu.get_barrier_semaphore()
        for p in range(N_DEV):
            @pl.when(p != my)
            def _():
                pl.semaphore_signal(
                    barrier, inc=1,
                    device_id=(p,), device_id_type=pl.DeviceIdType.MESH,
                )
        pl.semaphore_wait(barrier, N_DEV - 1)

        xf = x_ref[...].astype(jnp.float32)
        s1 = jnp.sum(xf, axis=-1)
        s2 = jnp.sum(xf * xf, axis=-1)
        stats_ref[my, 0, :, :] = s1
        stats_ref[my, 1, :, :] = s2

        for p in range(N_DEV):
            @pl.when(p != my)
            def _():
                pltpu.make_async_remote_copy(
                    src_ref=stats_ref.at[my],
                    dst_ref=stats_ref.at[my],
                    send_sem=send_sems.at[p],
                    recv_sem=recv_sems.at[my],
                    device_id=(p,),
                    device_id_type=pl.DeviceIdType.MESH,
                ).start()

        t32 = t_ref[...].astype(jnp.float32)
        scale = lax.dot_general(
            t32, wsc_ref[...].astype(jnp.float32),
            (((1,), (0,)), ((), ())), preferred_element_type=jnp.float32,
        )
        shift = lax.dot_general(
            t32, wsh_ref[...].astype(jnp.float32),
            (((1,), (0,)), ((), ())), preferred_element_type=jnp.float32,
        )

        for p in range(N_DEV):
            @pl.when(p != my)
            def _():
                rdma = pltpu.make_async_remote_copy(
                    src_ref=stats_ref.at[my],
                    dst_ref=stats_ref.at[p],
                    send_sem=send_sems.at[p],
                    recv_sem=recv_sems.at[p],
                    device_id=(p,),
                    device_id_type=pl.DeviceIdType.MESH,
                )
                rdma.wait_recv()
                rdma.wait_send()

        tot = stats_ref[...]
        s1t = jnp.sum(tot[:, 0], axis=0)
        s2t = jnp.sum(tot[:, 1], axis=0)
        mean = s1t / c_glob
        var = s2t / c_glob - mean * mean
        inv = lax.rsqrt(var + EPS)
        h = (xf - mean[:, :, None]) * inv[:, :, None]
        out = h * (1.0 + scale[:, None, :]) + shift[:, None, :]
        out_ref[...] = out.astype(out_ref.dtype)

    return pl.pallas_call(
        body,
        out_shape=jax.ShapeDtypeStruct((b, s, c_sh), x.dtype),
        in_specs=[pl.BlockSpec(memory_space=pltpu.VMEM)] * 4,
        out_specs=pl.BlockSpec(memory_space=pltpu.VMEM),
        scratch_shapes=[
            pltpu.VMEM((N_DEV, 2, b, s), jnp.float32),
            pltpu.SemaphoreType.DMA((N_DEV,)),
            pltpu.SemaphoreType.DMA((N_DEV,)),
        ],
        compiler_params=pltpu.CompilerParams(collective_id=0),
    )(x, t_emb, W_scale, W_shift)
